# baseline (speedup 1.0000x reference)
import functools

import jax
import jax.numpy as jnp
import numpy as np
from jax import lax
from jax.experimental import pallas as pl
from jax.experimental.pallas import tpu as pltpu, tpu_sc as plsc

N_SRC = 20000
N_QUERY = 2500
C_IN = 256
C_OUT = 512
K = 16

_PROJ_BLK = 400

_NC, _NS = 2, 16
_NW = _NC * _NS
_M_PAD = 2560
_Q_PER = _M_PAD // _NW
_NS_PAD = 20480
_N_BATCH = _NS_PAD // 16

_D2_BQ = 512
_D2_BS = 2048

_INF = np.float32(3.0e38)


def _d2_body(q_ref, st_ref, q2_ref, s2_ref, out_ref):
    out_ref[...] = (q2_ref[...] - 2.0 * jnp.dot(q_ref[...], st_ref[...],
                                                preferred_element_type=jnp.float32)
                    + s2_ref[...])


def _d2_all(xyz, n_xyz):
    q2 = jnp.sum(n_xyz * n_xyz, axis=1, keepdims=True)
    s2 = jnp.sum(xyz * xyz, axis=1)[None, :]
    xyz_t = jnp.pad(xyz.T, ((0, 0), (0, _NS_PAD - N_SRC)))
    s2 = jnp.pad(s2, ((0, 0), (0, _NS_PAD - N_SRC)), constant_values=_INF / 2)
    nq = jnp.pad(n_xyz, ((0, _M_PAD - N_QUERY), (0, 0)))
    q2 = jnp.pad(q2, ((0, _M_PAD - N_QUERY), (0, 0)))
    return pl.pallas_call(
        _d2_body,
        grid=(_M_PAD // _D2_BQ, _NS_PAD // _D2_BS),
        in_specs=[
            pl.BlockSpec((_D2_BQ, 3), lambda i, j: (i, 0)),
            pl.BlockSpec((3, _D2_BS), lambda i, j: (0, j)),
            pl.BlockSpec((_D2_BQ, 1), lambda i, j: (i, 0)),
            pl.BlockSpec((1, _D2_BS), lambda i, j: (0, j)),
        ],
        out_specs=pl.BlockSpec((_D2_BQ, _D2_BS), lambda i, j: (i, j)),
        out_shape=jax.ShapeDtypeStruct((_M_PAD, _NS_PAD), jnp.float32),
    )(nq, xyz_t, q2, s2)


def _proj_body(feats_ref, wt_ref, gamma_ref, beta_ref, out_ref):
    f = feats_ref[...]
    mean = jnp.mean(f, axis=1, keepdims=True)
    cent = f - mean
    var = jnp.mean(cent * cent, axis=1, keepdims=True)
    normed = cent * jax.lax.rsqrt(var + 1e-5) * gamma_ref[...] + beta_ref[...]
    out_ref[...] = jnp.dot(normed, wt_ref[...], preferred_element_type=jnp.float32)


def _project_all(feats, W, ln_gamma, ln_beta):
    wt = W.T
    gamma = ln_gamma.reshape(1, C_IN)
    beta = ln_beta.reshape(1, C_IN)
    grid = N_SRC // _PROJ_BLK
    return pl.pallas_call(
        _proj_body,
        grid=(grid,),
        in_specs=[
            pl.BlockSpec((_PROJ_BLK, C_IN), lambda i: (i, 0)),
            pl.BlockSpec((C_IN, C_OUT), lambda i: (0, 0)),
            pl.BlockSpec((1, C_IN), lambda i: (0, 0)),
            pl.BlockSpec((1, C_IN), lambda i: (0, 0)),
        ],
        out_specs=pl.BlockSpec((_PROJ_BLK, C_OUT), lambda i: (i, 0)),
        out_shape=jax.ShapeDtypeStruct((N_SRC, C_OUT), jnp.float32),
    )(feats, wt, gamma, beta)


_N_BLK = _NS_PAD // 256


def _topk_body(d2_hbm, out_hbm, rowa_v, rowb_v, minb_v, cand_v, fin_v, out_v,
               sema, semb):
    wid = lax.axis_index("s") * _NC + lax.axis_index("c")
    qbase = wid * _Q_PER
    iota = lax.iota(jnp.int32, 16)
    last = jnp.full((16,), 15, jnp.int32)
    dn = lax.GatherDimensionNumbers(offset_dims=(), collapsed_slice_dims=(0,),
                                    start_index_map=(0,))
    bcast_last = functools.partial(
        lax.gather, start_indices=last[:, None], dimension_numbers=dn,
        slice_sizes=(1,), mode=lax.GatherScatterMode.PROMISE_IN_BOUNDS)

    def process(row_v, qi):
        @plsc.parallel_loop(0, _N_BLK, unroll=2,
                            carry=(jnp.full((16,), _INF, jnp.float32),
                                   jnp.full((16,), _INF, jnp.float32)))
        def pa(b, vmins):
            vmin0, vmin1 = vmins
            base = b * 256
            vs = [row_v[pl.ds(base + 16 * u, 16)] for u in range(16)]
            while len(vs) > 2:
                vs = [jnp.minimum(vs[2 * i], vs[2 * i + 1])
                      for i in range(len(vs) // 2)]
            minb_v[pl.ds(b * 16, 16)] = jnp.minimum(vs[0], vs[1])
            return jnp.minimum(vmin0, vs[0]), jnp.minimum(vmin1, vs[1])

        vmin0, vmin1 = pa
        a0, _ = plsc.sort_key_val(vmin0, iota)
        a1, _ = plsc.sort_key_val(vmin1, iota, descending=True)
        low16 = jnp.minimum(a0, a1)
        tau0 = bcast_last(plsc.cummax(low16))
        tb = plsc.bitcast(tau0, jnp.int32)
        tau0p = plsc.bitcast(jnp.where(tau0 >= 0.0, tb + 1, tb - 1),
                             jnp.float32)

        def pb(b, cnt):
            bm = minb_v[pl.ds(b * 16, 16)]
            anyhit = plsc.all_reduce_population_count(bm < tau0p)[0] > 0

            def collect(cnt):
                base = b * 256
                for u in range(16):
                    dv = row_v[pl.ds(base + 16 * u, 16)]
                    msk = dv < tau0p
                    plsc.store_compressed(cand_v.at[pl.ds(cnt, 16)],
                                          iota + (base + 16 * u), mask=msk)
                    cnt = cnt + plsc.all_reduce_population_count(msk)[0]
                return cnt

            return lax.cond(anyhit, collect, lambda c: c, cnt)

        cnt = lax.fori_loop(0, _N_BLK, pb, jnp.int32(0))
        cand_v[pl.ds(cnt, 16)] = jnp.full((16,), _NS_PAD - 1, jnp.int32)

        def fm(j, carry):
            top_d, tau = carry
            ci = cand_v[pl.ds(j * 16, 16)]
            d = plsc.load_gather(row_v, [ci])
            hit = plsc.all_reduce_population_count(d < tau)[0] > 0

            def merge(args):
                top_d, _ = args
                d_dsc, _ = plsc.sort_key_val(d, ci, descending=True)
                m_d = jnp.minimum(top_d, d_dsc)
                new_d, _ = plsc.sort_key_val(m_d, iota)
                return new_d, bcast_last(new_d)

            return lax.cond(hit, merge, lambda a: a, (top_d, tau))

        init = (jnp.full((16,), _INF, jnp.float32),
                jnp.full((16,), _INF, jnp.float32))
        nv = (cnt + 15) // 16
        _, tauf = lax.fori_loop(0, nv, fm, init)

        def fe(j, carry):
            c_lt, c_eq = carry
            ci = cand_v[pl.ds(j * 16, 16)]
            d = plsc.load_gather(row_v, [ci])
            m_lt = d < tauf
            m_eq = d == tauf
            plsc.store_compressed(fin_v.at[pl.ds(c_lt, 16)], ci, mask=m_lt)
            plsc.store_compressed(cand_v.at[pl.ds(c_eq, 16)], ci, mask=m_eq)
            c_lt = c_lt + plsc.all_reduce_population_count(m_lt)[0]
            c_eq = c_eq + plsc.all_reduce_population_count(m_eq)[0]
            return c_lt, c_eq

        c_lt, _ = lax.fori_loop(0, nv, fe, (jnp.int32(0), jnp.int32(0)))
        fin_v[pl.ds(c_lt, 16)] = cand_v[pl.ds(0, 16)]
        out_v[pl.ds(qi * K, K)] = fin_v[pl.ds(0, 16)]

    pltpu.sync_copy(d2_hbm.at[qbase], rowa_v)

    def pair(i, _):
        q0 = 2 * i
        cpb = pltpu.async_copy(d2_hbm.at[qbase + q0 + 1], rowb_v, semb)
        process(rowa_v, q0)
        cpb.wait()
        nxt = jnp.minimum(qbase + q0 + 2, _M_PAD - 1)
        cpa = pltpu.async_copy(d2_hbm.at[nxt], rowa_v, sema)
        process(rowb_v, q0 + 1)
        cpa.wait()
        return 0

    lax.fori_loop(0, _Q_PER // 2, pair, 0)
    pltpu.sync_copy(out_v, out_hbm.at[pl.ds(qbase * K, _Q_PER * K)])


def _topk_idx(d2):
    mesh = plsc.VectorSubcoreMesh(core_axis_name="c", subcore_axis_name="s",
                                  num_cores=_NC, num_subcores=_NS)
    idx_flat = functools.partial(
        pl.kernel, mesh=mesh,
        compiler_params=pltpu.CompilerParams(needs_layout_passes=False),
        out_type=jax.ShapeDtypeStruct((_M_PAD * K,), jnp.int32),
        scratch_types=[
            pltpu.VMEM((_NS_PAD,), jnp.float32),
            pltpu.VMEM((_NS_PAD,), jnp.float32),
            pltpu.VMEM((_N_BLK * 16,), jnp.float32),
            pltpu.VMEM((_NS_PAD + 16,), jnp.int32),
            pltpu.VMEM((48,), jnp.int32),
            pltpu.VMEM((_Q_PER * K,), jnp.int32),
            pltpu.SemaphoreType.DMA,
            pltpu.SemaphoreType.DMA,
        ],
    )(_topk_body)(d2)
    return idx_flat.reshape(_M_PAD, K)[:N_QUERY]


def kernel(xyz, n_xyz, feats, ln_gamma, ln_beta, W):
    d2 = _d2_all(xyz, n_xyz)
    idx = _topk_idx(d2)
    proj = _project_all(feats, W, ln_gamma, ln_beta)
    pooled = jnp.max(proj[idx], axis=1)
    return pooled


# speedup vs baseline: 1.1247x; 1.1247x over previous
import functools

import jax
import jax.numpy as jnp
import numpy as np
from jax import lax
from jax.experimental import pallas as pl
from jax.experimental.pallas import tpu as pltpu, tpu_sc as plsc

N_SRC = 20000
N_QUERY = 2500
C_IN = 256
C_OUT = 512
K = 16

_PROJ_BLK = 400

_NC, _NS = 2, 16
_NW = _NC * _NS
_M_PAD = 2560
_Q_PER = _M_PAD // _NW
_NS_PAD = 20480
_N_BATCH = _NS_PAD // 16

_D2_BQ = 512
_D2_BS = 2048

_INF = np.float32(3.0e38)


def _d2_body(q_ref, st_ref, q2_ref, s2_ref, out_ref):
    out_ref[...] = (q2_ref[...] - 2.0 * jnp.dot(q_ref[...], st_ref[...],
                                                preferred_element_type=jnp.float32)
                    + s2_ref[...])


def _d2_all(xyz, n_xyz):
    q2 = jnp.sum(n_xyz * n_xyz, axis=1, keepdims=True)
    s2 = jnp.sum(xyz * xyz, axis=1)[None, :]
    xyz_t = jnp.pad(xyz.T, ((0, 0), (0, _NS_PAD - N_SRC)))
    s2 = jnp.pad(s2, ((0, 0), (0, _NS_PAD - N_SRC)), constant_values=_INF / 2)
    nq = jnp.pad(n_xyz, ((0, _M_PAD - N_QUERY), (0, 0)))
    q2 = jnp.pad(q2, ((0, _M_PAD - N_QUERY), (0, 0)))
    return pl.pallas_call(
        _d2_body,
        grid=(_M_PAD // _D2_BQ, _NS_PAD // _D2_BS),
        in_specs=[
            pl.BlockSpec((_D2_BQ, 3), lambda i, j: (i, 0)),
            pl.BlockSpec((3, _D2_BS), lambda i, j: (0, j)),
            pl.BlockSpec((_D2_BQ, 1), lambda i, j: (i, 0)),
            pl.BlockSpec((1, _D2_BS), lambda i, j: (0, j)),
        ],
        out_specs=pl.BlockSpec((_D2_BQ, _D2_BS), lambda i, j: (i, j)),
        out_shape=jax.ShapeDtypeStruct((_M_PAD, _NS_PAD), jnp.float32),
    )(nq, xyz_t, q2, s2)


def _proj_body(feats_ref, wt_ref, gamma_ref, beta_ref, out_ref):
    f = feats_ref[...]
    mean = jnp.mean(f, axis=1, keepdims=True)
    cent = f - mean
    var = jnp.mean(cent * cent, axis=1, keepdims=True)
    normed = cent * jax.lax.rsqrt(var + 1e-5) * gamma_ref[...] + beta_ref[...]
    out_ref[...] = jnp.dot(normed, wt_ref[...], preferred_element_type=jnp.float32)


def _project_all(feats, W, ln_gamma, ln_beta):
    wt = W.T
    gamma = ln_gamma.reshape(1, C_IN)
    beta = ln_beta.reshape(1, C_IN)
    grid = N_SRC // _PROJ_BLK
    return pl.pallas_call(
        _proj_body,
        grid=(grid,),
        in_specs=[
            pl.BlockSpec((_PROJ_BLK, C_IN), lambda i: (i, 0)),
            pl.BlockSpec((C_IN, C_OUT), lambda i: (0, 0)),
            pl.BlockSpec((1, C_IN), lambda i: (0, 0)),
            pl.BlockSpec((1, C_IN), lambda i: (0, 0)),
        ],
        out_specs=pl.BlockSpec((_PROJ_BLK, C_OUT), lambda i: (i, 0)),
        out_shape=jax.ShapeDtypeStruct((N_SRC, C_OUT), jnp.float32),
    )(feats, wt, gamma, beta)


_N_BLK = _NS_PAD // 256


def _topk_body(d2_hbm, proj_hbm, out_hbm, rowa_v, rowb_v, minb_v, cand_v,
               fin_v, idxa_v, idxb_v, growa_v, growb_v, pool_v,
               sema, semb, gsema, gsemb):
    wid = lax.axis_index("s") * _NC + lax.axis_index("c")
    qbase = wid * _Q_PER
    iota = lax.iota(jnp.int32, 16)
    last = jnp.full((16,), 15, jnp.int32)
    dn = lax.GatherDimensionNumbers(offset_dims=(), collapsed_slice_dims=(0,),
                                    start_index_map=(0,))
    bcast_last = functools.partial(
        lax.gather, start_indices=last[:, None], dimension_numbers=dn,
        slice_sizes=(1,), mode=lax.GatherScatterMode.PROMISE_IN_BOUNDS)

    def process(row_v, qi, idx_v, grow_v, gsem):
        @plsc.parallel_loop(0, _N_BLK, unroll=2,
                            carry=(jnp.full((16,), _INF, jnp.float32),
                                   jnp.full((16,), _INF, jnp.float32)))
        def pa(b, vmins):
            vmin0, vmin1 = vmins
            base = b * 256
            vs = [row_v[pl.ds(base + 16 * u, 16)] for u in range(16)]
            while len(vs) > 2:
                vs = [jnp.minimum(vs[2 * i], vs[2 * i + 1])
                      for i in range(len(vs) // 2)]
            minb_v[pl.ds(b * 16, 16)] = jnp.minimum(vs[0], vs[1])
            return jnp.minimum(vmin0, vs[0]), jnp.minimum(vmin1, vs[1])

        vmin0, vmin1 = pa
        a0, _ = plsc.sort_key_val(vmin0, iota)
        a1, _ = plsc.sort_key_val(vmin1, iota, descending=True)
        low16 = jnp.minimum(a0, a1)
        tau0 = bcast_last(plsc.cummax(low16))
        tb = plsc.bitcast(tau0, jnp.int32)
        tau0p = plsc.bitcast(jnp.where(tau0 >= 0.0, tb + 1, tb - 1),
                             jnp.float32)

        def pb(b, cnt):
            bm = minb_v[pl.ds(b * 16, 16)]
            anyhit = plsc.all_reduce_population_count(bm < tau0p)[0] > 0

            def collect(cnt):
                base = b * 256
                for u in range(16):
                    dv = row_v[pl.ds(base + 16 * u, 16)]
                    msk = dv < tau0p
                    plsc.store_compressed(cand_v.at[pl.ds(cnt, 16)],
                                          iota + (base + 16 * u), mask=msk)
                    cnt = cnt + plsc.all_reduce_population_count(msk)[0]
                return cnt

            return lax.cond(anyhit, collect, lambda c: c, cnt)

        cnt = lax.fori_loop(0, _N_BLK, pb, jnp.int32(0))
        cand_v[pl.ds(cnt, 16)] = jnp.full((16,), _NS_PAD - 1, jnp.int32)

        def fm(j, carry):
            top_d, tau = carry
            ci = cand_v[pl.ds(j * 16, 16)]
            d = plsc.load_gather(row_v, [ci])
            hit = plsc.all_reduce_population_count(d < tau)[0] > 0

            def merge(args):
                top_d, _ = args
                d_dsc, _ = plsc.sort_key_val(d, ci, descending=True)
                m_d = jnp.minimum(top_d, d_dsc)
                new_d, _ = plsc.sort_key_val(m_d, iota)
                return new_d, bcast_last(new_d)

            return lax.cond(hit, merge, lambda a: a, (top_d, tau))

        init = (jnp.full((16,), _INF, jnp.float32),
                jnp.full((16,), _INF, jnp.float32))
        nv = (cnt + 15) // 16
        _, tauf = lax.fori_loop(0, nv, fm, init)

        def fe(j, carry):
            c_lt, c_eq = carry
            ci = cand_v[pl.ds(j * 16, 16)]
            d = plsc.load_gather(row_v, [ci])
            m_lt = d < tauf
            m_eq = d == tauf
            plsc.store_compressed(fin_v.at[pl.ds(c_lt, 16)], ci, mask=m_lt)
            plsc.store_compressed(cand_v.at[pl.ds(c_eq, 16)], ci, mask=m_eq)
            c_lt = c_lt + plsc.all_reduce_population_count(m_lt)[0]
            c_eq = c_eq + plsc.all_reduce_population_count(m_eq)[0]
            return c_lt, c_eq

        c_lt, _ = lax.fori_loop(0, nv, fe, (jnp.int32(0), jnp.int32(0)))
        fin_v[pl.ds(c_lt, 16)] = cand_v[pl.ds(0, 16)]
        idx_v[pl.ds(0, 16)] = fin_v[pl.ds(0, 16)]
        pltpu.async_copy(proj_hbm.at[idx_v], grow_v, gsem)

    def finish(qi, idx_v, grow_v, gsem):
        pltpu.make_async_copy(proj_hbm.at[idx_v], grow_v, gsem).wait()

        @plsc.parallel_loop(0, C_OUT // 16)
        def _mx(c):
            vs = [grow_v[u, pl.ds(c * 16, 16)] for u in range(K)]
            while len(vs) > 1:
                vs = [jnp.maximum(vs[2 * i], vs[2 * i + 1])
                      for i in range(len(vs) // 2)]
            pool_v[pl.ds(qi * C_OUT + c * 16, 16)] = vs[0]

    pltpu.sync_copy(d2_hbm.at[qbase], rowa_v)

    def pair(i, _):
        q0 = 2 * i
        cpb = pltpu.async_copy(d2_hbm.at[qbase + q0 + 1], rowb_v, semb)
        process(rowa_v, q0, idxa_v, growa_v, gsema)

        @pl.when(i > 0)
        def _():
            finish(q0 - 1, idxb_v, growb_v, gsemb)

        cpb.wait()
        nxt = jnp.minimum(qbase + q0 + 2, _M_PAD - 1)
        cpa = pltpu.async_copy(d2_hbm.at[nxt], rowa_v, sema)
        process(rowb_v, q0 + 1, idxb_v, growb_v, gsemb)
        finish(q0, idxa_v, growa_v, gsema)
        cpa.wait()
        return 0

    lax.fori_loop(0, _Q_PER // 2, pair, 0)
    finish(_Q_PER - 1, idxb_v, growb_v, gsemb)
    pltpu.sync_copy(pool_v, out_hbm.at[pl.ds(qbase * C_OUT, _Q_PER * C_OUT)])


def _topk_pool(d2, proj):
    mesh = plsc.VectorSubcoreMesh(core_axis_name="c", subcore_axis_name="s",
                                  num_cores=_NC, num_subcores=_NS)
    pooled_flat = functools.partial(
        pl.kernel, mesh=mesh,
        compiler_params=pltpu.CompilerParams(needs_layout_passes=False),
        out_type=jax.ShapeDtypeStruct((_M_PAD * C_OUT,), jnp.float32),
        scratch_types=[
            pltpu.VMEM((_NS_PAD,), jnp.float32),
            pltpu.VMEM((_NS_PAD,), jnp.float32),
            pltpu.VMEM((_N_BLK * 16,), jnp.float32),
            pltpu.VMEM((_NS_PAD + 16,), jnp.int32),
            pltpu.VMEM((48,), jnp.int32),
            pltpu.VMEM((K,), jnp.int32),
            pltpu.VMEM((K,), jnp.int32),
            pltpu.VMEM((K, C_OUT), jnp.float32),
            pltpu.VMEM((K, C_OUT), jnp.float32),
            pltpu.VMEM((_Q_PER * C_OUT,), jnp.float32),
            pltpu.SemaphoreType.DMA,
            pltpu.SemaphoreType.DMA,
            pltpu.SemaphoreType.DMA,
            pltpu.SemaphoreType.DMA,
        ],
    )(_topk_body)(d2, proj)
    return pooled_flat.reshape(_M_PAD, C_OUT)[:N_QUERY]


def kernel(xyz, n_xyz, feats, ln_gamma, ln_beta, W):
    d2 = _d2_all(xyz, n_xyz)
    proj = _project_all(feats, W, ln_gamma, ln_beta)
    return _topk_pool(d2, proj)
